# baseline (device time: 152842 ns/iter reference)
import jax
import jax.numpy as jnp
from jax import lax
from jax.experimental import pallas as pl
from jax.experimental.pallas import tpu as pltpu

N_DEV = 8
BR = 64
D = 512
H = 1024


def kernel(x, Win0, Wout0, Win1, Wout1, Win2, Wout2):
    def body(x_ref, win0_ref, wout0_ref, win1_ref, wout1_ref, win2_ref,
             wout2_ref, out_ref, xg_ref, part_ref, xcur_ref, comm_ref,
             send_sems, recv_sems):
        my = lax.axis_index("i")
        left = lax.rem(my + (N_DEV - 1), N_DEV)
        right = lax.rem(my + 1, N_DEV)

        barrier_sem = pltpu.get_barrier_semaphore()
        for nbr in (left, right):
            pl.semaphore_signal(
                barrier_sem, inc=1,
                device_id=(nbr,), device_id_type=pl.DeviceIdType.MESH,
            )
        pl.semaphore_wait(barrier_sem, 2)

        xcur_ref[...] = x_ref[...]

        def hop(k):
            send_slot = k % 2
            recv_slot = (k + 1) % 2
            rdma = pltpu.make_async_remote_copy(
                src_ref=comm_ref.at[send_slot],
                dst_ref=comm_ref.at[recv_slot],
                send_sem=send_sems.at[send_slot],
                recv_sem=recv_sems.at[recv_slot],
                device_id=(right,),
                device_id_type=pl.DeviceIdType.MESH,
            )
            rdma.start()
            rdma.wait()
            return recv_slot

        weights = (
            (win0_ref, wout0_ref),
            (win1_ref, wout1_ref),
            (win2_ref, wout2_ref),
        )
        k = 0
        for win_ref, wout_ref in weights:
            xg_ref[pl.ds(my * BR, BR), :] = xcur_ref[...]
            comm_ref[k % 2] = xcur_ref[...]
            for h in range(N_DEV - 1):
                recv_slot = hop(k)
                origin = lax.rem(my + (N_DEV - 1 - h), N_DEV)
                xg_ref[pl.ds(origin * BR, BR), :] = comm_ref[recv_slot]
                k += 1

            hmat = jnp.maximum(
                jnp.dot(xg_ref[...], win_ref[...],
                        preferred_element_type=jnp.float32),
                0.0,
            )
            part_ref[...] = jnp.dot(hmat, wout_ref[...],
                                    preferred_element_type=jnp.float32)

            seed = lax.rem(my + (N_DEV - 1), N_DEV)
            comm_ref[k % 2] = part_ref[pl.ds(seed * BR, BR), :]
            for s in range(N_DEV - 1):
                recv_slot = hop(k)
                cidx = lax.rem(my + (N_DEV - 2 - s), N_DEV)
                chunk = part_ref[pl.ds(cidx * BR, BR), :]
                if s < N_DEV - 2:
                    comm_ref[recv_slot] = comm_ref[recv_slot] + chunk
                else:
                    xcur_ref[...] = comm_ref[recv_slot] + chunk
                k += 1

        out_ref[...] = xcur_ref[...]

    return pl.pallas_call(
        body,
        out_shape=jax.ShapeDtypeStruct((BR, D), jnp.float32),
        in_specs=[pl.BlockSpec(memory_space=pltpu.VMEM)] * 7,
        out_specs=pl.BlockSpec(memory_space=pltpu.VMEM),
        scratch_shapes=[
            pltpu.VMEM((N_DEV * BR, D), jnp.float32),
            pltpu.VMEM((N_DEV * BR, D), jnp.float32),
            pltpu.VMEM((BR, D), jnp.float32),
            pltpu.VMEM((2, BR, D), jnp.float32),
            pltpu.SemaphoreType.DMA((2,)),
            pltpu.SemaphoreType.DMA((2,)),
        ],
        compiler_params=pltpu.CompilerParams(collective_id=0),
    )(x, Win0, Wout0, Win1, Wout1, Win2, Wout2)


# device time: 70130 ns/iter; 2.1794x vs baseline; 2.1794x over previous
import jax
import jax.numpy as jnp
from jax import lax
from jax.experimental import pallas as pl
from jax.experimental.pallas import tpu as pltpu

N_DEV = 8
BR = 64
D = 512
H = 1024


def kernel(x, Win0, Wout0, Win1, Wout1, Win2, Wout2):
    def body(x_ref, win0_ref, wout0_ref, win1_ref, wout1_ref, win2_ref,
             wout2_ref, out_ref, xg_ref, part_ref, acc_ref, xcur_ref,
             b_send, b_recv, r_send, r_recv):
        my = lax.axis_index("i")

        def peer(o):
            return lax.rem(my + o, N_DEV)

        barrier_sem = pltpu.get_barrier_semaphore()
        for o in range(1, N_DEV):
            pl.semaphore_signal(
                barrier_sem, inc=1,
                device_id=(peer(o),), device_id_type=pl.DeviceIdType.MESH,
            )
        pl.semaphore_wait(barrier_sem, N_DEV - 1)

        xcur_ref[...] = x_ref[...]

        weights = (
            (win0_ref, wout0_ref),
            (win1_ref, wout1_ref),
            (win2_ref, wout2_ref),
        )
        for win_ref, wout_ref in weights:
            xg_ref[pl.ds(my * BR, BR), :] = xcur_ref[...]
            bcasts = []
            for o in range(1, N_DEV):
                rdma = pltpu.make_async_remote_copy(
                    src_ref=xcur_ref,
                    dst_ref=xg_ref.at[pl.ds(my * BR, BR), :],
                    send_sem=b_send.at[o],
                    recv_sem=b_recv.at[my],
                    device_id=(peer(o),),
                    device_id_type=pl.DeviceIdType.MESH,
                )
                rdma.start()
                bcasts.append(rdma)
            for o in range(1, N_DEV):
                s = peer(o)
                recv = pltpu.make_async_remote_copy(
                    src_ref=xcur_ref,
                    dst_ref=xg_ref.at[pl.ds(s * BR, BR), :],
                    send_sem=b_send.at[o],
                    recv_sem=b_recv.at[s],
                    device_id=(my,),
                    device_id_type=pl.DeviceIdType.MESH,
                )
                recv.wait_recv()
            for rdma in bcasts:
                rdma.wait_send()

            hmat = jnp.maximum(
                jnp.dot(xg_ref[...], win_ref[...],
                        preferred_element_type=jnp.float32),
                0.0,
            )
            part_ref[...] = jnp.dot(hmat, wout_ref[...],
                                    preferred_element_type=jnp.float32)

            acc_ref[my] = part_ref[pl.ds(my * BR, BR), :]
            reds = []
            for o in range(1, N_DEV):
                c = peer(o)
                rdma = pltpu.make_async_remote_copy(
                    src_ref=part_ref.at[pl.ds(c * BR, BR), :],
                    dst_ref=acc_ref.at[my],
                    send_sem=r_send.at[o],
                    recv_sem=r_recv.at[my],
                    device_id=(c,),
                    device_id_type=pl.DeviceIdType.MESH,
                )
                rdma.start()
                reds.append(rdma)
            for o in range(1, N_DEV):
                s = peer(o)
                recv = pltpu.make_async_remote_copy(
                    src_ref=part_ref.at[pl.ds(s * BR, BR), :],
                    dst_ref=acc_ref.at[s],
                    send_sem=r_send.at[o],
                    recv_sem=r_recv.at[s],
                    device_id=(my,),
                    device_id_type=pl.DeviceIdType.MESH,
                )
                recv.wait_recv()
            for rdma in reds:
                rdma.wait_send()

            xcur_ref[...] = jnp.sum(acc_ref[...], axis=0)

        out_ref[...] = xcur_ref[...]

    return pl.pallas_call(
        body,
        out_shape=jax.ShapeDtypeStruct((BR, D), jnp.float32),
        in_specs=[pl.BlockSpec(memory_space=pltpu.VMEM)] * 7,
        out_specs=pl.BlockSpec(memory_space=pltpu.VMEM),
        scratch_shapes=[
            pltpu.VMEM((N_DEV * BR, D), jnp.float32),
            pltpu.VMEM((N_DEV * BR, D), jnp.float32),
            pltpu.VMEM((N_DEV, BR, D), jnp.float32),
            pltpu.VMEM((BR, D), jnp.float32),
            pltpu.SemaphoreType.DMA((N_DEV,)),
            pltpu.SemaphoreType.DMA((N_DEV,)),
            pltpu.SemaphoreType.DMA((N_DEV,)),
            pltpu.SemaphoreType.DMA((N_DEV,)),
        ],
        compiler_params=pltpu.CompilerParams(collective_id=0),
    )(x, Win0, Wout0, Win1, Wout1, Win2, Wout2)


# device time: 60645 ns/iter; 2.5203x vs baseline; 1.1564x over previous
import jax
import jax.numpy as jnp
from jax import lax
from jax.experimental import pallas as pl
from jax.experimental.pallas import tpu as pltpu

N_DEV = 8
BR = 64
D = 512
H = 1024


def kernel(x, Win0, Wout0, Win1, Wout1, Win2, Wout2):
    def body(x_ref, win0_ref, wout0_ref, win1_ref, wout1_ref, win2_ref,
             wout2_ref, out_ref, xg_ref, part_ref, acc_ref, xcur_ref,
             b_send, b_recv, r_send, r_recv):
        my = lax.axis_index("i")

        def peer(o):
            return lax.rem(my + o, N_DEV)

        barrier_sem = pltpu.get_barrier_semaphore()
        for o in range(1, N_DEV):
            pl.semaphore_signal(
                barrier_sem, inc=1,
                device_id=(peer(o),), device_id_type=pl.DeviceIdType.MESH,
            )
        pl.semaphore_wait(barrier_sem, N_DEV - 1)

        xcur_ref[...] = x_ref[...]

        weights = (
            (win0_ref, wout0_ref),
            (win1_ref, wout1_ref),
            (win2_ref, wout2_ref),
        )
        for win_ref, wout_ref in weights:
            xg_ref[pl.ds(my * BR, BR), :] = xcur_ref[...]
            bcasts = []
            for o in range(1, N_DEV):
                rdma = pltpu.make_async_remote_copy(
                    src_ref=xcur_ref,
                    dst_ref=xg_ref.at[pl.ds(my * BR, BR), :],
                    send_sem=b_send.at[o],
                    recv_sem=b_recv.at[my],
                    device_id=(peer(o),),
                    device_id_type=pl.DeviceIdType.MESH,
                )
                rdma.start()
                bcasts.append(rdma)

            def wait_x_from(s):
                recv = pltpu.make_async_remote_copy(
                    src_ref=xcur_ref,
                    dst_ref=xg_ref.at[pl.ds(s * BR, BR), :],
                    send_sem=b_send.at[0],
                    recv_sem=b_recv.at[s],
                    device_id=(my,),
                    device_id_type=pl.DeviceIdType.MESH,
                )
                recv.wait_recv()

            reds = []
            for oa, ob in ((0, 7), (6, 5), (4, 3), (2, 1)):
                rows = []
                for o in (oa, ob):
                    if o == 0:
                        rows.append(xcur_ref[...])
                    else:
                        s = peer(o)
                        wait_x_from(s)
                        rows.append(xg_ref[pl.ds(s * BR, BR), :])
                x2 = jnp.concatenate(rows, axis=0)
                h2 = jnp.maximum(
                    jnp.dot(x2, win_ref[...],
                            preferred_element_type=jnp.float32),
                    0.0,
                )
                p2 = jnp.dot(h2, wout_ref[...],
                             preferred_element_type=jnp.float32)
                for j, o in enumerate((oa, ob)):
                    pj = p2[j * BR:(j + 1) * BR, :]
                    if o == 0:
                        acc_ref[my] = pj
                    else:
                        c = peer(o)
                        part_ref[pl.ds(c * BR, BR), :] = pj
                        rdma = pltpu.make_async_remote_copy(
                            src_ref=part_ref.at[pl.ds(c * BR, BR), :],
                            dst_ref=acc_ref.at[my],
                            send_sem=r_send.at[o],
                            recv_sem=r_recv.at[my],
                            device_id=(c,),
                            device_id_type=pl.DeviceIdType.MESH,
                        )
                        rdma.start()
                        reds.append(rdma)

            for o in range(1, N_DEV):
                s = peer(o)
                recv = pltpu.make_async_remote_copy(
                    src_ref=part_ref.at[pl.ds(s * BR, BR), :],
                    dst_ref=acc_ref.at[s],
                    send_sem=r_send.at[0],
                    recv_sem=r_recv.at[s],
                    device_id=(my,),
                    device_id_type=pl.DeviceIdType.MESH,
                )
                recv.wait_recv()
            for rdma in bcasts:
                rdma.wait_send()
            for rdma in reds:
                rdma.wait_send()

            xcur_ref[...] = jnp.sum(acc_ref[...], axis=0)

        out_ref[...] = xcur_ref[...]

    return pl.pallas_call(
        body,
        out_shape=jax.ShapeDtypeStruct((BR, D), jnp.float32),
        in_specs=[pl.BlockSpec(memory_space=pltpu.VMEM)] * 7,
        out_specs=pl.BlockSpec(memory_space=pltpu.VMEM),
        scratch_shapes=[
            pltpu.VMEM((N_DEV * BR, D), jnp.float32),
            pltpu.VMEM((N_DEV * BR, D), jnp.float32),
            pltpu.VMEM((N_DEV, BR, D), jnp.float32),
            pltpu.VMEM((BR, D), jnp.float32),
            pltpu.SemaphoreType.DMA((N_DEV,)),
            pltpu.SemaphoreType.DMA((N_DEV,)),
            pltpu.SemaphoreType.DMA((N_DEV,)),
            pltpu.SemaphoreType.DMA((N_DEV,)),
        ],
        compiler_params=pltpu.CompilerParams(collective_id=0),
    )(x, Win0, Wout0, Win1, Wout1, Win2, Wout2)


# device time: 45161 ns/iter; 3.3844x vs baseline; 1.3429x over previous
import jax
import jax.numpy as jnp
from jax import lax
from jax.experimental import pallas as pl
from jax.experimental.pallas import tpu as pltpu

N_DEV = 8
BR = 64
D = 512
H = 1024


def kernel(x, Win0, Wout0, Win1, Wout1, Win2, Wout2):
    def body(x_ref, win0_ref, wout0_ref, win1_ref, wout1_ref, win2_ref,
             wout2_ref, out_ref, xg_ref, part_ref, acc_ref, xcur_ref,
             xb_ref, pb_ref, b_send, b_recv, r_send, r_recv):
        my = lax.axis_index("i")

        def peer(o):
            return lax.rem(my + o, N_DEV)

        barrier_sem = pltpu.get_barrier_semaphore()
        for o in range(1, N_DEV):
            pl.semaphore_signal(
                barrier_sem, inc=1,
                device_id=(peer(o),), device_id_type=pl.DeviceIdType.MESH,
            )
        pl.semaphore_wait(barrier_sem, N_DEV - 1)

        xcur_ref[...] = x_ref[...]
        acc_ref[my] = jnp.zeros((BR, D), jnp.bfloat16)

        weights = (
            (win0_ref, wout0_ref),
            (win1_ref, wout1_ref),
            (win2_ref, wout2_ref),
        )
        for win_ref, wout_ref in weights:
            xb_ref[...] = xcur_ref[...].astype(jnp.bfloat16)
            bcasts = []
            for o in range(1, N_DEV):
                rdma = pltpu.make_async_remote_copy(
                    src_ref=xb_ref,
                    dst_ref=xg_ref.at[pl.ds(my * BR, BR), :],
                    send_sem=b_send.at[o],
                    recv_sem=b_recv.at[my],
                    device_id=(peer(o),),
                    device_id_type=pl.DeviceIdType.MESH,
                )
                rdma.start()
                bcasts.append(rdma)

            def wait_x_from(s):
                recv = pltpu.make_async_remote_copy(
                    src_ref=xb_ref,
                    dst_ref=xg_ref.at[pl.ds(s * BR, BR), :],
                    send_sem=b_send.at[0],
                    recv_sem=b_recv.at[s],
                    device_id=(my,),
                    device_id_type=pl.DeviceIdType.MESH,
                )
                recv.wait_recv()

            reds = []
            for oa, ob in ((0, 7), (6, 5), (4, 3), (2, 1)):
                rows = []
                for o in (oa, ob):
                    if o == 0:
                        rows.append(xcur_ref[...])
                    else:
                        s = peer(o)
                        wait_x_from(s)
                        rows.append(
                            xg_ref[pl.ds(s * BR, BR), :].astype(jnp.float32)
                        )
                x2 = jnp.concatenate(rows, axis=0)
                h2 = jnp.maximum(
                    jnp.dot(x2, win_ref[...],
                            preferred_element_type=jnp.float32),
                    0.0,
                )
                p2 = jnp.dot(h2, wout_ref[...],
                             preferred_element_type=jnp.float32)
                for j, o in enumerate((oa, ob)):
                    pj = p2[j * BR:(j + 1) * BR, :]
                    if o == 0:
                        part_ref[...] = pj
                    else:
                        c = peer(o)
                        pb_ref[pl.ds(c * BR, BR), :] = pj.astype(jnp.bfloat16)
                        rdma = pltpu.make_async_remote_copy(
                            src_ref=pb_ref.at[pl.ds(c * BR, BR), :],
                            dst_ref=acc_ref.at[my],
                            send_sem=r_send.at[o],
                            recv_sem=r_recv.at[my],
                            device_id=(c,),
                            device_id_type=pl.DeviceIdType.MESH,
                        )
                        rdma.start()
                        reds.append(rdma)

            for o in range(1, N_DEV):
                s = peer(o)
                recv = pltpu.make_async_remote_copy(
                    src_ref=pb_ref.at[pl.ds(s * BR, BR), :],
                    dst_ref=acc_ref.at[s],
                    send_sem=r_send.at[0],
                    recv_sem=r_recv.at[s],
                    device_id=(my,),
                    device_id_type=pl.DeviceIdType.MESH,
                )
                recv.wait_recv()
            for rdma in bcasts:
                rdma.wait_send()
            for rdma in reds:
                rdma.wait_send()

            xcur_ref[...] = part_ref[...] + jnp.sum(
                acc_ref[...].astype(jnp.float32), axis=0
            )

        out_ref[...] = xcur_ref[...]

    return pl.pallas_call(
        body,
        out_shape=jax.ShapeDtypeStruct((BR, D), jnp.float32),
        in_specs=[pl.BlockSpec(memory_space=pltpu.VMEM)] * 7,
        out_specs=pl.BlockSpec(memory_space=pltpu.VMEM),
        scratch_shapes=[
            pltpu.VMEM((N_DEV * BR, D), jnp.bfloat16),
            pltpu.VMEM((BR, D), jnp.float32),
            pltpu.VMEM((N_DEV, BR, D), jnp.bfloat16),
            pltpu.VMEM((BR, D), jnp.float32),
            pltpu.VMEM((BR, D), jnp.bfloat16),
            pltpu.VMEM((N_DEV * BR, D), jnp.bfloat16),
            pltpu.SemaphoreType.DMA((N_DEV,)),
            pltpu.SemaphoreType.DMA((N_DEV,)),
            pltpu.SemaphoreType.DMA((N_DEV,)),
            pltpu.SemaphoreType.DMA((N_DEV,)),
        ],
        compiler_params=pltpu.CompilerParams(collective_id=0),
    )(x, Win0, Wout0, Win1, Wout1, Win2, Wout2)
